# baseline (speedup 1.0000x reference)
import jax
import jax.numpy as jnp
from jax.experimental import pallas as pl

N_NODES = 10000
N_EDGES = 320000
NODE_DIM = 128
HID = 64


def _bn(x, g, b, eps=1e-5):
    m = jnp.mean(x, axis=0, keepdims=True)
    v = jnp.var(x, axis=0, keepdims=True)
    return g * (x - m) / jnp.sqrt(v + eps) + b


def kernel(x_edges, x_edges_values, x_nodes, x_nodes_coord, y_edges, edge_cw, edge_index, params):
    h = x_nodes.reshape(-1, NODE_DIM) @ params['coord_W']
    ev = x_edges_values.reshape(-1, 1)
    e = jnp.maximum(ev @ params['ee_W1'] + params['ee_b1'], 0.0)
    e = e @ params['ee_W2'] + params['ee_b2']
    ei = edge_index.reshape(2, -1)
    src, dst = ei[0], ei[1]
    for lp in params['layers']:
        Ue = e @ lp['eU_W'] + lp['eU_b']
        Vx = h @ lp['eV_W'] + lp['eV_b']
        e_tmp = Ue + Vx[src] + Vx[dst]
        gates = jax.nn.sigmoid(e_tmp)
        Ux = h @ lp['nU_W'] + lp['nU_b']
        Vx2 = h @ lp['nV_W'] + lp['nV_b']
        agg = jax.ops.segment_sum(gates * Vx2[dst], src, num_segments=N_NODES)
        h_tmp = Ux + agg
        e = e + jax.nn.relu(_bn(e_tmp, lp['bn_e_g'], lp['bn_e_b']))
        h = h + jax.nn.relu(_bn(h_tmp, lp['bn_h_g'], lp['bn_h_b']))
    y = e + h[src] + h[dst]
    for W, b in zip(params['mlp_Ws'][:-1], params['mlp_bs'][:-1]):
        y = jax.nn.relu(y @ W + b)
    y_pred = y @ params['mlp_Ws'][-1] + params['mlp_bs'][-1]
    logits = y_pred.reshape(-1)
    t = y_edges.reshape(-1).astype(jnp.float32)
    ce = jnp.maximum(logits, 0.0) - logits * t + jnp.log1p(jnp.exp(-jnp.abs(logits)))
    loss = jnp.mean(edge_cw * ce)
    return y_pred, loss


# speedup vs baseline: 2.2532x; 2.2532x over previous
import functools

import jax
import jax.numpy as jnp
from jax import lax
from jax.experimental import pallas as pl
from jax.experimental.pallas import tpu as pltpu
from jax.experimental.pallas import tpu_sc as plsc

N_NODES = 10000
N_EDGES = 320000
NODE_DIM = 128
HID = 64
F32 = jnp.float32

NC, NS, NL = 2, 16, 16
NW = NC * NS
EPT = N_EDGES // NW
CH = 80
NCHUNK = EPT // CH
NPAD = 10240
RPT = NPAD // NS

EB = 4000
EG = N_EDGES // EB


def _node_embed_k(xn_ref, w_ref, o_ref):
    o_ref[...] = jnp.dot(xn_ref[...], w_ref[...], preferred_element_type=F32)


def _edge_embed_k(ev_ref, w1_ref, b1_ref, w2_ref, b2_ref, o_ref):
    t = jnp.maximum(ev_ref[...] * w1_ref[...] + b1_ref[...], 0.0)
    o_ref[...] = jnp.dot(t, w2_ref[...], preferred_element_type=F32) + b2_ref[...]


def _node_dense_k(hp_ref, evw, evb, nuw, nub, nvw, nvb, vxd_ref, ux_ref):
    h = hp_ref[...][:, 0:HID]
    vx = jnp.dot(h, evw[...], preferred_element_type=F32) + evb[...]
    vx2 = jnp.dot(h, nvw[...], preferred_element_type=F32) + nvb[...]
    vxd_ref[...] = jnp.concatenate([vx, vx2], axis=1)
    ux_ref[...] = jnp.dot(h, nuw[...], preferred_element_type=F32) + nub[...]


def _edge_dense_k(e_ref, w_ref, b_ref, o_ref):
    o_ref[...] = jnp.dot(e_ref[...], w_ref[...], preferred_element_type=F32) + b_ref[...]


def _estats_k(x_ref, o_ref):
    @pl.when(pl.program_id(0) == 0)
    def _():
        o_ref[...] = jnp.zeros_like(o_ref)

    x = x_ref[...]
    s = jnp.sum(x, axis=0, keepdims=True)
    ss = jnp.sum(x * x, axis=0, keepdims=True)
    o_ref[...] += jnp.concatenate(
        [s, ss, jnp.zeros((6, HID), F32)], axis=0)


def _eresid_k(e_ref, et_ref, st_ref, g_ref, b_ref, o_ref):
    m = st_ref[0:1, :] / N_EDGES
    v = st_ref[1:2, :] / N_EDGES - m * m
    et = et_ref[...]
    bn = g_ref[...] * (et - m) * lax.rsqrt(v + 1e-5) + b_ref[...]
    o_ref[...] = e_ref[...] + jnp.maximum(bn, 0.0)


def _node_update_k(ux_ref, agg_ref, hp_ref, g_ref, b_ref, o_ref):
    ht = ux_ref[...] + agg_ref[0] + agg_ref[1]
    m = jnp.mean(ht, axis=0, keepdims=True)
    v = jnp.mean(ht * ht, axis=0, keepdims=True) - m * m
    bn = g_ref[...] * (ht - m) * lax.rsqrt(v + 1e-5) + b_ref[...]
    hn = hp_ref[...][:, 0:HID] + jnp.maximum(bn, 0.0)
    o_ref[...] = jnp.concatenate([hn, jnp.zeros((N_NODES, HID), F32)], axis=1)


def _mlp_loss_k(y_ref, t_ref, w0, b0, w1, b1, w2, b2, yp_ref, ls_ref):
    y1 = jnp.maximum(jnp.dot(y_ref[...], w0[...], preferred_element_type=F32) + b0[...], 0.0)
    y2 = jnp.maximum(jnp.dot(y1, w1[...], preferred_element_type=F32) + b1[...], 0.0)
    yp = jnp.dot(y2, w2[...], preferred_element_type=F32) + b2[...]
    yp_ref[...] = yp
    t = t_ref[...]
    ce = jnp.maximum(yp, 0.0) - yp * t + jnp.log1p(jnp.exp(-jnp.abs(yp)))

    @pl.when(pl.program_id(0) == 0)
    def _():
        ls_ref[...] = jnp.zeros_like(ls_ref)

    ls_ref[...] += jnp.sum(ce, axis=0, keepdims=True)


def _full(shape):
    return pl.BlockSpec(shape, lambda i: tuple(0 for _ in shape))


def _eblk(width):
    return pl.BlockSpec((EB, width), lambda i: (i, 0))


def _sc_edge_body(ue_hbm, vxd_hbm, src_hbm, dst_hbm,
                  etmp_hbm, msg_hbm,
                  idx_s, idx_d, g1_v, g23_v, ue_v, et_v, msg_v, sem):
    c = lax.axis_index("c")
    s = lax.axis_index("s")
    wid = s * NC + c
    base = wid * EPT

    def chunk(j, carry):
        off = base + j * CH
        pltpu.sync_copy(src_hbm.at[pl.ds(off, CH)], idx_s)
        pltpu.sync_copy(dst_hbm.at[pl.ds(off, CH)], idx_d)
        d1 = pltpu.async_copy(vxd_hbm.at[idx_s], g1_v, sem)
        d2 = pltpu.async_copy(vxd_hbm.at[idx_d], g23_v, sem)
        d3 = pltpu.async_copy(ue_hbm.at[pl.ds(off, CH), :], ue_v, sem)
        d1.wait()
        d2.wait()
        d3.wait()

        def row(i, acc):
            for k in range(HID // NL):
                sl = pl.ds(k * NL, NL)
                a = ue_v[i, sl] + g1_v[i, sl] + g23_v[i, sl]
                et_v[i, sl] = a
                d = 1.0 + jnp.exp(-a)
                r = 1.0 / d
                r = r * (2.0 - d * r)
                msg_v[i, sl] = g23_v[i, pl.ds(HID + k * NL, NL)] * r
            return acc

        lax.fori_loop(0, CH, row, 0)
        pltpu.sync_copy(et_v, etmp_hbm.at[pl.ds(off, CH), :])
        pltpu.sync_copy(msg_v, msg_hbm.at[pl.ds(off, CH), :])
        return carry

    lax.fori_loop(0, NCHUNK, chunk, 0)


def _sc_y_body(e_hbm, hp_hbm, src_hbm, dst_hbm, y_hbm,
               idx_s, idx_d, g1_v, g2_v, e_v, sem):
    c = lax.axis_index("c")
    s = lax.axis_index("s")
    wid = s * NC + c
    base = wid * EPT

    def chunk(j, carry):
        off = base + j * CH
        pltpu.sync_copy(src_hbm.at[pl.ds(off, CH)], idx_s)
        pltpu.sync_copy(dst_hbm.at[pl.ds(off, CH)], idx_d)
        d1 = pltpu.async_copy(hp_hbm.at[idx_s], g1_v, sem)
        d2 = pltpu.async_copy(hp_hbm.at[idx_d], g2_v, sem)
        d3 = pltpu.async_copy(e_hbm.at[pl.ds(off, CH), :], e_v, sem)
        d1.wait()
        d2.wait()
        d3.wait()

        def row(i, acc):
            for k in range(HID // NL):
                sl = pl.ds(k * NL, NL)
                e_v[i, sl] = e_v[i, sl] + g1_v[i, sl] + g2_v[i, sl]
            return acc

        lax.fori_loop(0, CH, row, 0)
        pltpu.sync_copy(e_v, y_hbm.at[pl.ds(off, CH), :])
        return carry

    lax.fori_loop(0, NCHUNK, chunk, 0)


_SC_MESH = plsc.VectorSubcoreMesh(core_axis_name="c", subcore_axis_name="s")

_sc_edge = functools.partial(
    pl.kernel,
    out_type=[jax.ShapeDtypeStruct((N_EDGES, HID), F32),
              jax.ShapeDtypeStruct((N_EDGES, HID), F32)],
    mesh=_SC_MESH,
    scratch_types=[
        pltpu.VMEM((CH,), jnp.int32),
        pltpu.VMEM((CH,), jnp.int32),
        pltpu.VMEM((CH, 2 * HID), F32),
        pltpu.VMEM((CH, 2 * HID), F32),
        pltpu.VMEM((CH, HID), F32),
        pltpu.VMEM((CH, HID), F32),
        pltpu.VMEM((CH, HID), F32),
        pltpu.SemaphoreType.DMA,
    ],
)(_sc_edge_body)

_sc_y = functools.partial(
    pl.kernel,
    out_type=jax.ShapeDtypeStruct((N_EDGES, HID), F32),
    mesh=_SC_MESH,
    scratch_types=[
        pltpu.VMEM((CH,), jnp.int32),
        pltpu.VMEM((CH,), jnp.int32),
        pltpu.VMEM((CH, 2 * HID), F32),
        pltpu.VMEM((CH, 2 * HID), F32),
        pltpu.VMEM((CH, HID), F32),
        pltpu.SemaphoreType.DMA,
    ],
)(_sc_y_body)


def kernel(x_edges, x_edges_values, x_nodes, x_nodes_coord, y_edges, edge_cw,
           edge_index, params):
    xn = x_nodes.reshape(-1, NODE_DIM).astype(F32)
    ev = x_edges_values.reshape(-1, 1).astype(F32)
    ei = edge_index.reshape(2, -1).astype(jnp.int32)
    src, dst = ei[0], ei[1]
    t = y_edges.reshape(-1, 1).astype(F32)

    def r2(b):
        return b.reshape(1, -1)

    cw128 = jnp.concatenate(
        [params['coord_W'], jnp.zeros((NODE_DIM, HID), F32)], axis=1)
    h = pl.pallas_call(
        _node_embed_k,
        out_shape=jax.ShapeDtypeStruct((N_NODES, 2 * HID), F32),
    )(xn, cw128)

    e = pl.pallas_call(
        _edge_embed_k,
        grid=(EG,),
        in_specs=[_eblk(1), _full((1, HID)), _full((1, HID)),
                  _full((HID, HID)), _full((1, HID))],
        out_specs=_eblk(HID),
        out_shape=jax.ShapeDtypeStruct((N_EDGES, HID), F32),
    )(ev, params['ee_W1'], r2(params['ee_b1']), params['ee_W2'], r2(params['ee_b2']))

    for lp in params['layers']:
        vxd, ux = pl.pallas_call(
            _node_dense_k,
            out_shape=[jax.ShapeDtypeStruct((N_NODES, 2 * HID), F32),
                       jax.ShapeDtypeStruct((N_NODES, HID), F32)],
        )(h, lp['eV_W'], r2(lp['eV_b']), lp['nU_W'], r2(lp['nU_b']),
          lp['nV_W'], r2(lp['nV_b']))

        ue = pl.pallas_call(
            _edge_dense_k,
            grid=(EG,),
            in_specs=[_eblk(HID), _full((HID, HID)), _full((1, HID))],
            out_specs=_eblk(HID),
            out_shape=jax.ShapeDtypeStruct((N_EDGES, HID), F32),
        )(e, lp['eU_W'], r2(lp['eU_b']))

        etmp, msg = _sc_edge(ue, vxd, src, dst)
        agg0 = jax.ops.segment_sum(msg, src, num_segments=N_NODES)
        agg = jnp.stack([agg0, jnp.zeros_like(agg0)]).reshape(NC, N_NODES, HID)

        estats = pl.pallas_call(
            _estats_k,
            grid=(EG,),
            in_specs=[_eblk(HID)],
            out_specs=_full((8, HID)),
            out_shape=jax.ShapeDtypeStruct((8, HID), F32),
        )(etmp)

        e = pl.pallas_call(
            _eresid_k,
            grid=(EG,),
            in_specs=[_eblk(HID), _eblk(HID), _full((8, HID)),
                      _full((1, HID)), _full((1, HID))],
            out_specs=_eblk(HID),
            out_shape=jax.ShapeDtypeStruct((N_EDGES, HID), F32),
        )(e, etmp, estats, r2(lp['bn_e_g']), r2(lp['bn_e_b']))

        h = pl.pallas_call(
            _node_update_k,
            out_shape=jax.ShapeDtypeStruct((N_NODES, 2 * HID), F32),
        )(ux, agg, h, r2(lp['bn_h_g']), r2(lp['bn_h_b']))

    y = _sc_y(e, h, src, dst)

    y_pred, loss_sum = pl.pallas_call(
        _mlp_loss_k,
        grid=(EG,),
        in_specs=[_eblk(HID), _eblk(1),
                  _full((HID, HID)), _full((1, HID)),
                  _full((HID, HID)), _full((1, HID)),
                  _full((HID, 1)), _full((1, 1))],
        out_specs=[_eblk(1), _full((1, 1))],
        out_shape=[jax.ShapeDtypeStruct((N_EDGES, 1), F32),
                   jax.ShapeDtypeStruct((1, 1), F32)],
    )(y, t, params['mlp_Ws'][0], r2(params['mlp_bs'][0]),
      params['mlp_Ws'][1], r2(params['mlp_bs'][1]),
      params['mlp_Ws'][2], params['mlp_bs'][2].reshape(1, 1))

    loss = (edge_cw[0] * loss_sum[0, 0] / N_EDGES).astype(F32)
    return y_pred, loss
